# Initial kernel scaffold; baseline (speedup 1.0000x reference)
import jax
import jax.numpy as jnp
from jax import lax
from jax.experimental import pallas as pl
from jax.experimental.pallas import tpu as pltpu
from jax.experimental.pallas import tpu_sc as plsc

BATCH = 16384
TOKENS = 20
D = 64
NC, NS = 2, 16
NW = NC * NS
DOCS_PER_W = BATCH // NW
CHUNK_DOCS = 32
ROWS_PER_CHUNK = CHUNK_DOCS * TOKENS
IDX_PER_DMA = 128
DMAS_PER_CHUNK = ROWS_PER_CHUNK // IDX_PER_DMA
CHUNKS = DOCS_PER_W // CHUNK_DOCS


def _pool_sc_kernel(idx_hbm, table_hbm, out_hbm, idx_v, rows_v, out_v, sem):
    wid = lax.axis_index("s") * NC + lax.axis_index("c")

    @pl.loop(0, CHUNKS)
    def _chunk(c):
        doc_base = wid * DOCS_PER_W + c * CHUNK_DOCS
        idx_row0 = doc_base * TOKENS // IDX_PER_DMA
        pltpu.sync_copy(idx_hbm.at[pl.ds(idx_row0, DMAS_PER_CHUNK)], idx_v)
        for j in range(DMAS_PER_CHUNK):
            pltpu.async_copy(
                table_hbm.at[idx_v.at[j]],
                rows_v.at[pl.ds(j * IDX_PER_DMA, IDX_PER_DMA)],
                sem,
            )
        for j in range(DMAS_PER_CHUNK):
            pltpu.make_async_copy(
                table_hbm.at[idx_v.at[j]],
                rows_v.at[pl.ds(j * IDX_PER_DMA, IDX_PER_DMA)],
                sem,
            ).wait()

        @pl.loop(0, CHUNK_DOCS)
        def _doc(d):
            row0 = d * TOKENS

            def body(t, accs):
                return tuple(
                    accs[k] + rows_v[row0 + t, pl.ds(k * 16, 16)]
                    for k in range(4)
                )

            zero = jnp.zeros((16,), jnp.float32)
            accs = lax.fori_loop(0, TOKENS, body, (zero, zero, zero, zero))
            for k in range(4):
                out_v[d, pl.ds(k * 16, 16)] = accs[k]

        pltpu.sync_copy(out_v, out_hbm.at[pl.ds(doc_base, CHUNK_DOCS)])


def _pool_sc(idx2d, table):
    mesh = plsc.VectorSubcoreMesh(
        core_axis_name="c", subcore_axis_name="s", num_cores=NC, num_subcores=NS
    )
    f = pl.kernel(
        _pool_sc_kernel,
        out_type=jax.ShapeDtypeStruct((BATCH, D), jnp.float32),
        mesh=mesh,
        scratch_types=[
            pltpu.VMEM((DMAS_PER_CHUNK, IDX_PER_DMA), jnp.int32),
            pltpu.VMEM((ROWS_PER_CHUNK, D), jnp.float32),
            pltpu.VMEM((CHUNK_DOCS, D), jnp.float32),
            pltpu.SemaphoreType.DMA,
        ],
    )
    return f(idx2d, table)


def _mm_kernel(x_ref, w_ref, o_ref):
    o_ref[...] = lax.dot_general(
        x_ref[...],
        w_ref[...],
        (((1,), (1,)), ((), ())),
        preferred_element_type=jnp.float32,
        precision=lax.Precision.HIGHEST,
    )


def _linear_tc(pooled, W):
    blk = 1024
    return pl.pallas_call(
        _mm_kernel,
        out_shape=jax.ShapeDtypeStruct((BATCH, D), jnp.float32),
        grid=(BATCH // blk,),
        in_specs=[
            pl.BlockSpec((blk, D), lambda i: (i, 0)),
            pl.BlockSpec((D, D), lambda i: (0, 0)),
        ],
        out_specs=pl.BlockSpec((blk, D), lambda i: (i, 0)),
    )(pooled, W)


def kernel(document, table, W):
    idx2d = document[:, :TOKENS].reshape(BATCH * TOKENS // IDX_PER_DMA, IDX_PER_DMA)
    pooled = _pool_sc(idx2d, table)
    return _linear_tc(pooled, W)


# speedup vs baseline: 2.1278x; 2.1278x over previous
import jax
import jax.numpy as jnp
from jax import lax
from jax.experimental import pallas as pl
from jax.experimental.pallas import tpu as pltpu
from jax.experimental.pallas import tpu_sc as plsc

BATCH = 16384
TOKENS = 20
D = 64
NC, NS = 2, 16
NW = NC * NS
DOCS_PER_W = BATCH // NW
CHUNK_DOCS = 32
ROWS_PER_CHUNK = CHUNK_DOCS * TOKENS
IDX_PER_DMA = 128
DMAS_PER_CHUNK = ROWS_PER_CHUNK // IDX_PER_DMA
CHUNKS = DOCS_PER_W // CHUNK_DOCS


def _pool_sc_kernel(idx_hbm, table_hbm, out_hbm, idx_v, rows_v, out_v, sem):
    wid = lax.axis_index("s") * NC + lax.axis_index("c")

    @pl.loop(0, CHUNKS)
    def _chunk(c):
        g = wid * CHUNKS + c
        doc_base = g * CHUNK_DOCS
        pltpu.sync_copy(idx_hbm.at[g], idx_v)
        for j in range(DMAS_PER_CHUNK):
            pltpu.async_copy(
                table_hbm.at[idx_v.at[j]],
                rows_v.at[pl.ds(j * IDX_PER_DMA, IDX_PER_DMA)],
                sem,
            )
        for j in range(DMAS_PER_CHUNK):
            pltpu.make_async_copy(
                table_hbm.at[idx_v.at[j]],
                rows_v.at[pl.ds(j * IDX_PER_DMA, IDX_PER_DMA)],
                sem,
            ).wait()

        @pl.loop(0, CHUNK_DOCS)
        def _doc(d):
            row0 = d * TOKENS

            def body(t, accs):
                return tuple(
                    accs[k] + rows_v[row0 + t, pl.ds(k * 16, 16)]
                    for k in range(4)
                )

            zero = jnp.zeros((16,), jnp.float32)
            accs = lax.fori_loop(0, TOKENS, body, (zero, zero, zero, zero))
            for k in range(4):
                out_v[d, pl.ds(k * 16, 16)] = accs[k]

        pltpu.sync_copy(out_v, out_hbm.at[pl.ds(doc_base, CHUNK_DOCS)])


def _pool_sc(idx2d, table):
    mesh = plsc.VectorSubcoreMesh(
        core_axis_name="c", subcore_axis_name="s", num_cores=NC, num_subcores=NS
    )
    f = pl.kernel(
        _pool_sc_kernel,
        out_type=jax.ShapeDtypeStruct((BATCH, D), jnp.float32),
        mesh=mesh,
        scratch_types=[
            pltpu.VMEM((DMAS_PER_CHUNK, IDX_PER_DMA), jnp.int32),
            pltpu.VMEM((ROWS_PER_CHUNK, D), jnp.float32),
            pltpu.VMEM((CHUNK_DOCS, D), jnp.float32),
            pltpu.SemaphoreType.DMA,
        ],
        compiler_params=pltpu.CompilerParams(use_tc_tiling_on_sc=False),
    )
    return f(idx2d, table)


def _mm_kernel(x_ref, w_ref, o_ref):
    o_ref[...] = lax.dot_general(
        x_ref[...],
        w_ref[...],
        (((1,), (1,)), ((), ())),
        preferred_element_type=jnp.float32,
        precision=lax.Precision.HIGHEST,
    )


def _linear_tc(pooled, W):
    blk = 1024
    return pl.pallas_call(
        _mm_kernel,
        out_shape=jax.ShapeDtypeStruct((BATCH, D), jnp.float32),
        grid=(BATCH // blk,),
        in_specs=[
            pl.BlockSpec((blk, D), lambda i: (i, 0)),
            pl.BlockSpec((D, D), lambda i: (0, 0)),
        ],
        out_specs=pl.BlockSpec((blk, D), lambda i: (i, 0)),
    )(pooled, W)


def kernel(document, table, W):
    idx3d = document[:, :TOKENS].reshape(
        NW * CHUNKS, DMAS_PER_CHUNK, IDX_PER_DMA
    )
    pooled = _pool_sc(idx3d, table)
    return _linear_tc(pooled, W)
